# baseline (speedup 1.0000x reference)
import functools

import jax
import jax.numpy as jnp
from jax import lax
from jax.experimental import pallas as pl
from jax.experimental.pallas import tpu as pltpu
from jax.experimental.pallas import tpu_sc as plsc

N = 10000
E = 320000
E2 = E // 2
D_IN = 128
H = 8
D = 16
HD = H * D
CLIP = 5.0

NC = 2
NS = 16
NW = NC * NS
EPW = E2 // NW
ACC_W = HD + 16
RCHUNK = 40
NRC = N // RCHUNK
NRC_PT = -(-NRC // NS)

CA = 40
CHUNKS_A = EPW // CA
CB = 40
CHUNKS_B = EPW // CB


def _matmul_bias(x, w, b, block_rows):
    m, k = x.shape
    f = w.shape[1]

    def body(x_ref, w_ref, b_ref, o_ref):
        o_ref[...] = (
            jnp.dot(x_ref[...], w_ref[...], preferred_element_type=jnp.float32)
            + b_ref[0:1, :]
        )

    return pl.pallas_call(
        body,
        grid=(m // block_rows,),
        in_specs=[
            pl.BlockSpec((block_rows, k), lambda i: (i, 0)),
            pl.BlockSpec((k, f), lambda i: (0, 0)),
            pl.BlockSpec((8, f), lambda i: (0, 0)),
        ],
        out_specs=pl.BlockSpec((block_rows, f), lambda i: (i, 0)),
        out_shape=jax.ShapeDtypeStruct((m, f), jnp.float32),
    )(x, w, jnp.broadcast_to(b, (8, f)))


def _edge_elementwise(g, edge_feats, we, be, half, eo_prev):
    bn = 8000
    nblk = E2 // bn
    off = half * nblk

    def body(g_ref, x_ref, w_ref, b_ref, *refs):
        eo_ref, t_ref = refs[-2], refs[-1]
        pe = (jnp.dot(x_ref[...], w_ref[...],
                      preferred_element_type=jnp.float32) + b_ref[0:1, :])
        eprod = g_ref[...] * pe
        eo_ref[...] = eprod
        row = lax.broadcasted_iota(jnp.int32, (HD, 16), 0)
        col = lax.broadcasted_iota(jnp.int32, (HD, 16), 1)
        bsel = (row // D == col).astype(jnp.float32)
        sums = jnp.dot(eprod, bsel, preferred_element_type=jnp.float32)
        t_ref[...] = jnp.exp(jnp.clip(sums, -CLIP, CLIP))

    in_specs = [
        pl.BlockSpec((bn, HD), lambda i: (i, 0)),
        pl.BlockSpec((bn, D_IN), lambda i: (i + off, 0)),
        pl.BlockSpec((D_IN, HD), lambda i: (0, 0)),
        pl.BlockSpec((8, HD), lambda i: (0, 0)),
    ]
    args = [g, edge_feats, we, jnp.broadcast_to(be, (8, HD))]
    aliases = {}
    if half == 1:
        in_specs.append(pl.BlockSpec(memory_space=pl.ANY))
        args.append(eo_prev)
        aliases = {4: 0}

    return pl.pallas_call(
        body,
        grid=(nblk,),
        in_specs=in_specs,
        out_specs=[
            pl.BlockSpec((bn, HD), lambda i: (i + off, 0)),
            pl.BlockSpec((bn, 16), lambda i: (i, 0)),
        ],
        out_shape=[
            jax.ShapeDtypeStruct((E, HD), jnp.float32),
            jax.ShapeDtypeStruct((E2, 16), jnp.float32),
        ],
        input_output_aliases=aliases,
    )(*args)


def _combine(p0, p1):
    bn = 2000

    def body(p0_ref, p1_ref, o_ref):
        p = p0_ref[0] + p0_ref[1] + p1_ref[0] + p1_ref[1]
        wv = p[:, :HD]
        z = p[:, HD:]
        col = lax.broadcasted_iota(jnp.int32, (16, HD), 1)
        row = lax.broadcasted_iota(jnp.int32, (16, HD), 0)
        sel = ((col // D == row) & (row < H)).astype(jnp.float32)
        zrep = jnp.dot(z, sel, preferred_element_type=jnp.float32)
        o_ref[...] = wv / (zrep + 1e-6)

    return pl.pallas_call(
        body,
        grid=(N // bn,),
        in_specs=[
            pl.BlockSpec((NC, bn, ACC_W), lambda i: (0, i, 0)),
            pl.BlockSpec((NC, bn, ACC_W), lambda i: (0, i, 0)),
        ],
        out_specs=pl.BlockSpec((bn, HD), lambda i: (i, 0)),
        out_shape=jax.ShapeDtypeStruct((N, HD), jnp.float32),
    )(p0, p1)


def _sc_mesh():
    return plsc.VectorSubcoreMesh(
        core_axis_name="c", subcore_axis_name="s",
        num_cores=NC, num_subcores=NS)


@functools.lru_cache(maxsize=2)
def _make_sc_score_kernel(half):
    @functools.partial(
        pl.kernel,
        out_type=jax.ShapeDtypeStruct((E2, HD), jnp.float32),
        mesh=_sc_mesh(),
        compiler_params=pltpu.CompilerParams(
            use_tc_tiling_on_sc=False, needs_layout_passes=False),
        scratch_types=[
            [pltpu.VMEM((CA,), jnp.int32) for _ in range(2)],
            [pltpu.VMEM((CA,), jnp.int32) for _ in range(2)],
            [pltpu.VMEM((CA, HD), jnp.float32) for _ in range(2)],
            [pltpu.VMEM((CA, HD), jnp.float32) for _ in range(2)],
            [pltpu.VMEM((CA, HD), jnp.float32) for _ in range(2)],
            [pltpu.SemaphoreType.DMA for _ in range(2)],
            [pltpu.SemaphoreType.DMA for _ in range(2)],
            [pltpu.SemaphoreType.DMA for _ in range(2)],
            [pltpu.SemaphoreType.DMA for _ in range(2)],
            [pltpu.SemaphoreType.DMA for _ in range(2)],
        ],
    )
    def sc_score_kernel(kh, qh, ei, g_hbm,
                        src_v, dst_v, k_v, q_v, g_v,
                        sem_is, sem_id, sem_k, sem_q, sem_g):
        cid = lax.axis_index("c")
        sid = lax.axis_index("s")
        wbase = (cid * NS + sid) * EPW
        ebase = half * E2 + wbase

        def issue_idx(n, p):
            eb = pl.multiple_of(ebase + n * CA, 8)
            pltpu.async_copy(ei.at[0, pl.ds(eb, CA)], src_v[p], sem_is[p])
            pltpu.async_copy(ei.at[1, pl.ds(eb, CA)], dst_v[p], sem_id[p])

        def drain_idx(p):
            pltpu.make_async_copy(
                ei.at[0, pl.ds(0, CA)], src_v[p], sem_is[p]).wait()
            pltpu.make_async_copy(
                ei.at[1, pl.ds(0, CA)], dst_v[p], sem_id[p]).wait()

        def issue_gathers(p):
            pltpu.async_copy(kh.at[src_v[p]], k_v[p], sem_k[p])
            pltpu.async_copy(qh.at[dst_v[p]], q_v[p], sem_q[p])

        def drain_gathers(p):
            pltpu.make_async_copy(kh.at[src_v[p]], k_v[p], sem_k[p]).wait()
            pltpu.make_async_copy(qh.at[dst_v[p]], q_v[p], sem_q[p]).wait()

        def drain_gout(p):
            pltpu.make_async_copy(
                g_v[p], g_hbm.at[pl.ds(0, CA), :], sem_g[p]).wait()

        def compute_chunk(p):
            kb, qb, gb = k_v[p], q_v[p], g_v[p]

            @plsc.parallel_loop(0, CA, unroll=4)
            def _(c):
                for h in range(H):
                    sl = pl.ds(h * D, D)
                    kq = kb[c, sl] * qb[c, sl]
                    gb[c, sl] = jnp.clip(kq * 0.25, -CLIP, CLIP)

        issue_idx(0, 0)
        issue_idx(1, 1)
        drain_idx(0)
        issue_gathers(0)

        def pair_body(gi, _):
            for b in range(2):
                p = b
                q = 1 - b
                j = 2 * gi + b

                @pl.when(j < CHUNKS_A)
                def _():
                    gob = pl.multiple_of(wbase + j * CA, 8)
                    drain_gathers(p)

                    @pl.when(j >= 2)
                    def _():
                        drain_gout(p)

                    @pl.when(j + 1 < CHUNKS_A)
                    def _():
                        drain_idx(q)
                        issue_gathers(q)

                    @pl.when(j + 2 < CHUNKS_A)
                    def _():
                        issue_idx(j + 2, p)

                    compute_chunk(p)
                    pltpu.async_copy(
                        g_v[p], g_hbm.at[pl.ds(gob, CA), :], sem_g[p])
            return 0

        lax.fori_loop(0, -(-CHUNKS_A // 2), pair_body, 0)
        drain_gout(0)
        drain_gout(1)

    return sc_score_kernel


@functools.lru_cache(maxsize=2)
def _make_sc_scatter_kernel(half):
    @functools.partial(
        pl.kernel,
        out_type=jax.ShapeDtypeStruct((NC, N, ACC_W), jnp.float32),
        mesh=_sc_mesh(),
        compiler_params=pltpu.CompilerParams(
            use_tc_tiling_on_sc=False, needs_layout_passes=False),
        scratch_types=[
            [pltpu.VMEM((CB,), jnp.int32) for _ in range(2)],
            [pltpu.VMEM((CB,), jnp.int32) for _ in range(2)],
            [pltpu.VMEM((CB,), jnp.int32) for _ in range(2)],
            [pltpu.VMEM((CB, HD), jnp.float32) for _ in range(2)],
            [pltpu.VMEM((CB, 16), jnp.float32) for _ in range(2)],
            [pltpu.VMEM((CB, ACC_W), jnp.float32) for _ in range(2)],
            pltpu.VMEM((RCHUNK, ACC_W), jnp.float32),
            pltpu.VMEM_SHARED((N, ACC_W), jnp.float32),
            pltpu.SemaphoreType.DMA,
            [pltpu.SemaphoreType.DMA for _ in range(2)],
            [pltpu.SemaphoreType.DMA for _ in range(2)],
            [pltpu.SemaphoreType.DMA for _ in range(2)],
            [pltpu.SemaphoreType.DMA for _ in range(2)],
            [pltpu.SemaphoreType.DMA for _ in range(2)],
        ],
    )
    def sc_scatter_kernel(vh, t, ei, wv_hbm,
                          src_v, dst_v, dst_sc, v_v, t_v, wv_v,
                          zb, acc, sem_io,
                          sem_is, sem_id, sem_v, sem_t, sem_sc):
        cid = lax.axis_index("c")
        sid = lax.axis_index("s")
        wbase = (cid * NS + sid) * EPW
        ebase = half * E2 + wbase

        zeros16 = jnp.zeros((16,), jnp.float32)

        def zero_row(r, _):
            for h in range(ACC_W // D):
                zb[r, pl.ds(h * D, D)] = zeros16
            return 0

        lax.fori_loop(0, RCHUNK, zero_row, 0)

        def init_body(j, _):
            row = j * NS + sid

            @pl.when(row < NRC)
            def _():
                base = pl.multiple_of(row * RCHUNK, 8)
                pltpu.async_copy(zb, acc.at[pl.ds(base, RCHUNK), :], sem_io)
            return 0

        lax.fori_loop(0, NRC_PT, init_body, 0)

        def init_drain(j, _):
            row = j * NS + sid

            @pl.when(row < NRC)
            def _():
                pltpu.make_async_copy(
                    zb, acc.at[pl.ds(0, RCHUNK), :], sem_io).wait()
            return 0

        lax.fori_loop(0, NRC_PT, init_drain, 0)
        plsc.subcore_barrier()

        def issue_idx(n, p):
            eb = pl.multiple_of(ebase + n * CB, 8)
            pltpu.async_copy(ei.at[0, pl.ds(eb, CB)], src_v[p], sem_is[p])
            pltpu.async_copy(ei.at[1, pl.ds(eb, CB)], dst_v[p], sem_id[p])

        def drain_idx(p):
            pltpu.make_async_copy(
                ei.at[0, pl.ds(0, CB)], src_v[p], sem_is[p]).wait()
            pltpu.make_async_copy(
                ei.at[1, pl.ds(0, CB)], dst_v[p], sem_id[p]).wait()

        def issue_gathers(n, p):
            tb = pl.multiple_of(wbase + n * CB, 8)
            pltpu.async_copy(vh.at[src_v[p]], v_v[p], sem_v[p])
            pltpu.async_copy(t.at[pl.ds(tb, CB), :], t_v[p], sem_t[p])

        def drain_gathers(p):
            pltpu.make_async_copy(vh.at[src_v[p]], v_v[p], sem_v[p]).wait()
            pltpu.make_async_copy(
                t.at[pl.ds(0, CB), :], t_v[p], sem_t[p]).wait()

        def drain_scat(p):
            pltpu.make_async_copy(
                wv_v[p], acc.at[dst_sc[p]], sem_sc[p]).wait()

        def compute_chunk(p):
            vb, tb, wvb = v_v[p], t_v[p], wv_v[p]

            @plsc.parallel_loop(0, CB, unroll=4)
            def _(c):
                trow = tb[c, :]
                for h in range(H):
                    th = trow.at[
                        jnp.full((16,), h, jnp.int32)].get(
                            mode="promise_in_bounds")
                    wvb[c, pl.ds(h * D, D)] = vb[c, pl.ds(h * D, D)] * th
                wvb[c, pl.ds(HD, 16)] = trow

        issue_idx(0, 0)
        issue_idx(1, 1)
        drain_idx(0)
        issue_gathers(0, 0)

        def pair_body(gi, _):
            for b in range(2):
                p = b
                q = 1 - b
                j = 2 * gi + b

                @pl.when(j < CHUNKS_B)
                def _():
                    drain_gathers(p)

                    @pl.when(j >= 2)
                    def _():
                        drain_scat(p)

                    for o in (0, 16, CB - 16):
                        dst_sc[p][pl.ds(o, 16)] = dst_v[p][pl.ds(o, 16)]

                    @pl.when(j + 1 < CHUNKS_B)
                    def _():
                        drain_idx(q)
                        issue_gathers(j + 1, q)

                    @pl.when(j + 2 < CHUNKS_B)
                    def _():
                        issue_idx(j + 2, p)

                    compute_chunk(p)
                    pltpu.async_copy(
                        wv_v[p], acc.at[dst_sc[p]], sem_sc[p], add=True)
            return 0

        lax.fori_loop(0, -(-CHUNKS_B // 2), pair_body, 0)
        drain_scat(0)
        drain_scat(1)
        plsc.subcore_barrier()

        def out_body(j, _):
            row = j * NS + sid

            @pl.when(row < NRC)
            def _():
                base = pl.multiple_of(row * RCHUNK, 8)
                pltpu.sync_copy(acc.at[pl.ds(base, RCHUNK), :], zb)
                pltpu.sync_copy(zb, wv_hbm.at[cid, pl.ds(base, RCHUNK), :])
            return 0

        lax.fori_loop(0, NRC_PT, out_body, 0)

    return sc_scatter_kernel


def kernel(node_feats, edge_feats, edge_index, Wq, bq, Wk, bk, Wv, bv, We, be):
    w_qkv = jnp.concatenate([Wq, Wk, Wv], axis=1)
    b_qkv = jnp.concatenate([bq, bk, bv], axis=0)
    qkv = _matmul_bias(node_feats, w_qkv, b_qkv, 2000)
    q_h = qkv[:, :HD]
    k_h = qkv[:, HD:2 * HD]
    v_h = qkv[:, 2 * HD:]

    g0 = _make_sc_score_kernel(0)(k_h, q_h, edge_index)
    g1 = _make_sc_score_kernel(1)(k_h, q_h, edge_index)
    eo0, t0 = _edge_elementwise(g0, edge_feats, We, be, 0, None)
    e_out, t1 = _edge_elementwise(g1, edge_feats, We, be, 1, eo0)
    p0 = _make_sc_scatter_kernel(0)(v_h, t0, edge_index)
    p1 = _make_sc_scatter_kernel(1)(v_h, t1, edge_index)
    h_out = _combine(p0, p1)
    return (h_out.reshape(N, H, D), e_out.reshape(E, H, D))


# speedup vs baseline: 1.0058x; 1.0058x over previous
import functools

import jax
import jax.numpy as jnp
from jax import lax
from jax.experimental import pallas as pl
from jax.experimental.pallas import tpu as pltpu
from jax.experimental.pallas import tpu_sc as plsc

N = 10000
E = 320000
E2 = E // 2
D_IN = 128
H = 8
D = 16
HD = H * D
CLIP = 5.0

NC = 2
NS = 16
NW = NC * NS
EPW = E2 // NW
ACC_W = HD + 16
RCHUNK = 40
NRC = N // RCHUNK
NRC_PT = -(-NRC // NS)

CA = 40
CHUNKS_A = EPW // CA
CB = 40
CHUNKS_B = EPW // CB


def _matmul_bias(x, w, b, block_rows):
    m, k = x.shape
    f = w.shape[1]

    def body(x_ref, w_ref, b_ref, o_ref):
        o_ref[...] = (
            jnp.dot(x_ref[...], w_ref[...], preferred_element_type=jnp.float32)
            + b_ref[0:1, :]
        )

    return pl.pallas_call(
        body,
        grid=(m // block_rows,),
        in_specs=[
            pl.BlockSpec((block_rows, k), lambda i: (i, 0)),
            pl.BlockSpec((k, f), lambda i: (0, 0)),
            pl.BlockSpec((8, f), lambda i: (0, 0)),
        ],
        out_specs=pl.BlockSpec((block_rows, f), lambda i: (i, 0)),
        out_shape=jax.ShapeDtypeStruct((m, f), jnp.float32),
    )(x, w, jnp.broadcast_to(b, (8, f)))


def _edge_elementwise(g, edge_feats, we, be, half, eo_prev):
    bn = 10000
    nblk = E2 // bn
    off = half * nblk

    def body(g_ref, x_ref, w_ref, b_ref, *refs):
        eo_ref, t_ref = refs[-2], refs[-1]
        pe = (jnp.dot(x_ref[...], w_ref[...],
                      preferred_element_type=jnp.float32) + b_ref[0:1, :])
        eprod = g_ref[...] * pe
        eo_ref[...] = eprod
        row = lax.broadcasted_iota(jnp.int32, (HD, 16), 0)
        col = lax.broadcasted_iota(jnp.int32, (HD, 16), 1)
        bsel = (row // D == col).astype(jnp.float32)
        sums = jnp.dot(eprod, bsel, preferred_element_type=jnp.float32)
        t_ref[...] = jnp.exp(jnp.clip(sums, -CLIP, CLIP))

    in_specs = [
        pl.BlockSpec((bn, HD), lambda i: (i, 0)),
        pl.BlockSpec((bn, D_IN), lambda i: (i + off, 0)),
        pl.BlockSpec((D_IN, HD), lambda i: (0, 0)),
        pl.BlockSpec((8, HD), lambda i: (0, 0)),
    ]
    args = [g, edge_feats, we, jnp.broadcast_to(be, (8, HD))]
    aliases = {}
    if half == 1:
        in_specs.append(pl.BlockSpec(memory_space=pl.ANY))
        args.append(eo_prev)
        aliases = {4: 0}

    return pl.pallas_call(
        body,
        grid=(nblk,),
        in_specs=in_specs,
        out_specs=[
            pl.BlockSpec((bn, HD), lambda i: (i + off, 0)),
            pl.BlockSpec((bn, 16), lambda i: (i, 0)),
        ],
        out_shape=[
            jax.ShapeDtypeStruct((E, HD), jnp.float32),
            jax.ShapeDtypeStruct((E2, 16), jnp.float32),
        ],
        input_output_aliases=aliases,
    )(*args)


def _combine(p0, p1):
    bn = 2000

    def body(p0_ref, p1_ref, o_ref):
        p = p0_ref[0] + p0_ref[1] + p1_ref[0] + p1_ref[1]
        wv = p[:, :HD]
        z = p[:, HD:]
        col = lax.broadcasted_iota(jnp.int32, (16, HD), 1)
        row = lax.broadcasted_iota(jnp.int32, (16, HD), 0)
        sel = ((col // D == row) & (row < H)).astype(jnp.float32)
        zrep = jnp.dot(z, sel, preferred_element_type=jnp.float32)
        o_ref[...] = wv / (zrep + 1e-6)

    return pl.pallas_call(
        body,
        grid=(N // bn,),
        in_specs=[
            pl.BlockSpec((NC, bn, ACC_W), lambda i: (0, i, 0)),
            pl.BlockSpec((NC, bn, ACC_W), lambda i: (0, i, 0)),
        ],
        out_specs=pl.BlockSpec((bn, HD), lambda i: (i, 0)),
        out_shape=jax.ShapeDtypeStruct((N, HD), jnp.float32),
    )(p0, p1)


def _sc_mesh():
    return plsc.VectorSubcoreMesh(
        core_axis_name="c", subcore_axis_name="s",
        num_cores=NC, num_subcores=NS)


@functools.lru_cache(maxsize=2)
def _make_sc_score_kernel(half):
    @functools.partial(
        pl.kernel,
        out_type=jax.ShapeDtypeStruct((E2, HD), jnp.float32),
        mesh=_sc_mesh(),
        compiler_params=pltpu.CompilerParams(
            use_tc_tiling_on_sc=False, needs_layout_passes=False),
        scratch_types=[
            [pltpu.VMEM((CA,), jnp.int32) for _ in range(2)],
            [pltpu.VMEM((CA,), jnp.int32) for _ in range(2)],
            [pltpu.VMEM((CA, HD), jnp.float32) for _ in range(2)],
            [pltpu.VMEM((CA, HD), jnp.float32) for _ in range(2)],
            [pltpu.VMEM((CA, HD), jnp.float32) for _ in range(2)],
            [pltpu.SemaphoreType.DMA for _ in range(2)],
            [pltpu.SemaphoreType.DMA for _ in range(2)],
            [pltpu.SemaphoreType.DMA for _ in range(2)],
            [pltpu.SemaphoreType.DMA for _ in range(2)],
            [pltpu.SemaphoreType.DMA for _ in range(2)],
        ],
    )
    def sc_score_kernel(kh, qh, ei, g_hbm,
                        src_v, dst_v, k_v, q_v, g_v,
                        sem_is, sem_id, sem_k, sem_q, sem_g):
        cid = lax.axis_index("c")
        sid = lax.axis_index("s")
        wbase = (cid * NS + sid) * EPW
        ebase = half * E2 + wbase

        def issue_idx(n, p):
            eb = pl.multiple_of(ebase + n * CA, 8)
            pltpu.async_copy(ei.at[0, pl.ds(eb, CA)], src_v[p], sem_is[p])
            pltpu.async_copy(ei.at[1, pl.ds(eb, CA)], dst_v[p], sem_id[p])

        def drain_idx(p):
            pltpu.make_async_copy(
                ei.at[0, pl.ds(0, CA)], src_v[p], sem_is[p]).wait()
            pltpu.make_async_copy(
                ei.at[1, pl.ds(0, CA)], dst_v[p], sem_id[p]).wait()

        def issue_gathers(p):
            pltpu.async_copy(kh.at[src_v[p]], k_v[p], sem_k[p])
            pltpu.async_copy(qh.at[dst_v[p]], q_v[p], sem_q[p])

        def drain_gathers(p):
            pltpu.make_async_copy(kh.at[src_v[p]], k_v[p], sem_k[p]).wait()
            pltpu.make_async_copy(qh.at[dst_v[p]], q_v[p], sem_q[p]).wait()

        def drain_gout(p):
            pltpu.make_async_copy(
                g_v[p], g_hbm.at[pl.ds(0, CA), :], sem_g[p]).wait()

        def compute_chunk(p):
            kb, qb, gb = k_v[p], q_v[p], g_v[p]

            @plsc.parallel_loop(0, CA, unroll=4)
            def _(c):
                for h in range(H):
                    sl = pl.ds(h * D, D)
                    kq = kb[c, sl] * qb[c, sl]
                    gb[c, sl] = jnp.clip(kq * 0.25, -CLIP, CLIP)

        issue_idx(0, 0)
        issue_idx(1, 1)
        drain_idx(0)
        issue_gathers(0)

        def pair_body(gi, _):
            for b in range(2):
                p = b
                q = 1 - b
                j = 2 * gi + b

                @pl.when(j < CHUNKS_A)
                def _():
                    gob = pl.multiple_of(wbase + j * CA, 8)
                    drain_gathers(p)

                    @pl.when(j >= 2)
                    def _():
                        drain_gout(p)

                    @pl.when(j + 1 < CHUNKS_A)
                    def _():
                        drain_idx(q)
                        issue_gathers(q)

                    @pl.when(j + 2 < CHUNKS_A)
                    def _():
                        issue_idx(j + 2, p)

                    compute_chunk(p)
                    pltpu.async_copy(
                        g_v[p], g_hbm.at[pl.ds(gob, CA), :], sem_g[p])
            return 0

        lax.fori_loop(0, -(-CHUNKS_A // 2), pair_body, 0)
        drain_gout(0)
        drain_gout(1)

    return sc_score_kernel


@functools.lru_cache(maxsize=2)
def _make_sc_scatter_kernel(half):
    @functools.partial(
        pl.kernel,
        out_type=jax.ShapeDtypeStruct((NC, N, ACC_W), jnp.float32),
        mesh=_sc_mesh(),
        compiler_params=pltpu.CompilerParams(
            use_tc_tiling_on_sc=False, needs_layout_passes=False),
        scratch_types=[
            [pltpu.VMEM((CB,), jnp.int32) for _ in range(2)],
            [pltpu.VMEM((CB,), jnp.int32) for _ in range(2)],
            [pltpu.VMEM((CB,), jnp.int32) for _ in range(2)],
            [pltpu.VMEM((CB, HD), jnp.float32) for _ in range(2)],
            [pltpu.VMEM((CB, 16), jnp.float32) for _ in range(2)],
            [pltpu.VMEM((CB, ACC_W), jnp.float32) for _ in range(2)],
            [pltpu.VMEM((RCHUNK, ACC_W), jnp.float32) for _ in range(2)],
            pltpu.VMEM_SHARED((N, ACC_W), jnp.float32),
            [pltpu.SemaphoreType.DMA for _ in range(2)],
            [pltpu.SemaphoreType.DMA for _ in range(2)],
            [pltpu.SemaphoreType.DMA for _ in range(2)],
            [pltpu.SemaphoreType.DMA for _ in range(2)],
            [pltpu.SemaphoreType.DMA for _ in range(2)],
            [pltpu.SemaphoreType.DMA for _ in range(2)],
        ],
    )
    def sc_scatter_kernel(vh, t, ei, wv_hbm,
                          src_v, dst_v, dst_sc, v_v, t_v, wv_v,
                          zb, acc, sem_io,
                          sem_is, sem_id, sem_v, sem_t, sem_sc):
        cid = lax.axis_index("c")
        sid = lax.axis_index("s")
        wbase = (cid * NS + sid) * EPW
        ebase = half * E2 + wbase

        zeros16 = jnp.zeros((16,), jnp.float32)

        def zero_row(r, _):
            for h in range(ACC_W // D):
                zb[0][r, pl.ds(h * D, D)] = zeros16
            return 0

        lax.fori_loop(0, RCHUNK, zero_row, 0)

        def init_body(j, _):
            row = j * NS + sid

            @pl.when(row < NRC)
            def _():
                base = pl.multiple_of(row * RCHUNK, 8)
                pltpu.async_copy(
                    zb[0], acc.at[pl.ds(base, RCHUNK), :], sem_io[0])
            return 0

        lax.fori_loop(0, NRC_PT, init_body, 0)

        def init_drain(j, _):
            row = j * NS + sid

            @pl.when(row < NRC)
            def _():
                pltpu.make_async_copy(
                    zb[0], acc.at[pl.ds(0, RCHUNK), :], sem_io[0]).wait()
            return 0

        lax.fori_loop(0, NRC_PT, init_drain, 0)
        plsc.subcore_barrier()

        def issue_idx(n, p):
            eb = pl.multiple_of(ebase + n * CB, 8)
            pltpu.async_copy(ei.at[0, pl.ds(eb, CB)], src_v[p], sem_is[p])
            pltpu.async_copy(ei.at[1, pl.ds(eb, CB)], dst_v[p], sem_id[p])

        def drain_idx(p):
            pltpu.make_async_copy(
                ei.at[0, pl.ds(0, CB)], src_v[p], sem_is[p]).wait()
            pltpu.make_async_copy(
                ei.at[1, pl.ds(0, CB)], dst_v[p], sem_id[p]).wait()

        def issue_gathers(n, p):
            tb = pl.multiple_of(wbase + n * CB, 8)
            pltpu.async_copy(vh.at[src_v[p]], v_v[p], sem_v[p])
            pltpu.async_copy(t.at[pl.ds(tb, CB), :], t_v[p], sem_t[p])

        def drain_gathers(p):
            pltpu.make_async_copy(vh.at[src_v[p]], v_v[p], sem_v[p]).wait()
            pltpu.make_async_copy(
                t.at[pl.ds(0, CB), :], t_v[p], sem_t[p]).wait()

        def drain_scat(p):
            pltpu.make_async_copy(
                wv_v[p], acc.at[dst_sc[p]], sem_sc[p]).wait()

        def compute_chunk(p):
            vb, tb, wvb = v_v[p], t_v[p], wv_v[p]

            @plsc.parallel_loop(0, CB, unroll=4)
            def _(c):
                trow = tb[c, :]
                for h in range(H):
                    th = trow.at[
                        jnp.full((16,), h, jnp.int32)].get(
                            mode="promise_in_bounds")
                    wvb[c, pl.ds(h * D, D)] = vb[c, pl.ds(h * D, D)] * th
                wvb[c, pl.ds(HD, 16)] = trow

        issue_idx(0, 0)
        issue_idx(1, 1)
        drain_idx(0)
        issue_gathers(0, 0)

        def pair_body(gi, _):
            for b in range(2):
                p = b
                q = 1 - b
                j = 2 * gi + b

                @pl.when(j < CHUNKS_B)
                def _():
                    drain_gathers(p)

                    @pl.when(j >= 2)
                    def _():
                        drain_scat(p)

                    for o in (0, 16, CB - 16):
                        dst_sc[p][pl.ds(o, 16)] = dst_v[p][pl.ds(o, 16)]

                    @pl.when(j + 1 < CHUNKS_B)
                    def _():
                        drain_idx(q)
                        issue_gathers(j + 1, q)

                    @pl.when(j + 2 < CHUNKS_B)
                    def _():
                        issue_idx(j + 2, p)

                    compute_chunk(p)
                    pltpu.async_copy(
                        wv_v[p], acc.at[dst_sc[p]], sem_sc[p], add=True)
            return 0

        lax.fori_loop(0, -(-CHUNKS_B // 2), pair_body, 0)
        drain_scat(0)
        drain_scat(1)
        plsc.subcore_barrier()

        def out_drain(j):
            pltpu.make_async_copy(
                zb[j % 2], wv_hbm.at[cid, pl.ds(0, RCHUNK), :],
                sem_io[j % 2]).wait()

        for j in range(NRC_PT):
            row = j * NS + sid

            @pl.when(row < NRC)
            def _():
                if j >= 2:
                    out_drain(j - 2)
                base = pl.multiple_of(row * RCHUNK, 8)
                pltpu.sync_copy(acc.at[pl.ds(base, RCHUNK), :], zb[j % 2])
                pltpu.async_copy(
                    zb[j % 2], wv_hbm.at[cid, pl.ds(base, RCHUNK), :],
                    sem_io[j % 2])

        for j in range(NRC_PT - 4, NRC_PT):
            row = j * NS + sid
            nxt = (j + 2) * NS + sid
            issued = row < NRC
            not_drained = (j + 2 >= NRC_PT) | (nxt >= NRC)

            @pl.when(issued & not_drained)
            def _():
                out_drain(j)

    return sc_scatter_kernel


def kernel(node_feats, edge_feats, edge_index, Wq, bq, Wk, bk, Wv, bv, We, be):
    w_qkv = jnp.concatenate([Wq, Wk, Wv], axis=1)
    b_qkv = jnp.concatenate([bq, bk, bv], axis=0)
    qkv = _matmul_bias(node_feats, w_qkv, b_qkv, 2000)
    q_h = qkv[:, :HD]
    k_h = qkv[:, HD:2 * HD]
    v_h = qkv[:, 2 * HD:]

    g0 = _make_sc_score_kernel(0)(k_h, q_h, edge_index)
    g1 = _make_sc_score_kernel(1)(k_h, q_h, edge_index)
    eo0, t0 = _edge_elementwise(g0, edge_feats, We, be, 0, None)
    e_out, t1 = _edge_elementwise(g1, edge_feats, We, be, 1, eo0)
    p0 = _make_sc_scatter_kernel(0)(v_h, t0, edge_index)
    p1 = _make_sc_scatter_kernel(1)(v_h, t1, edge_index)
    h_out = _combine(p0, p1)
    return (h_out.reshape(N, H, D), e_out.reshape(E, H, D))


# speedup vs baseline: 1.0821x; 1.0758x over previous
import functools

import jax
import jax.numpy as jnp
from jax import lax
from jax.experimental import pallas as pl
from jax.experimental.pallas import tpu as pltpu
from jax.experimental.pallas import tpu_sc as plsc

N = 10000
E = 320000
E2 = E // 2
D_IN = 128
H = 8
D = 16
HD = H * D
CLIP = 5.0

NC = 2
NS = 16
NW = NC * NS
EPW = E2 // NW
ACC_W = HD + 16
RCHUNK = 40
NRC = N // RCHUNK
NRC_PT = -(-NRC // NS)

CA = 40
CHUNKS_A = EPW // CA
CB = 40
CHUNKS_B = EPW // CB


def _matmul_bias(x, w, b, block_rows):
    m, k = x.shape
    f = w.shape[1]

    def body(x_ref, w_ref, b_ref, o_ref):
        o_ref[...] = (
            jnp.dot(x_ref[...], w_ref[...], preferred_element_type=jnp.float32)
            + b_ref[0:1, :]
        )

    return pl.pallas_call(
        body,
        grid=(m // block_rows,),
        in_specs=[
            pl.BlockSpec((block_rows, k), lambda i: (i, 0)),
            pl.BlockSpec((k, f), lambda i: (0, 0)),
            pl.BlockSpec((8, f), lambda i: (0, 0)),
        ],
        out_specs=pl.BlockSpec((block_rows, f), lambda i: (i, 0)),
        out_shape=jax.ShapeDtypeStruct((m, f), jnp.float32),
    )(x, w, jnp.broadcast_to(b, (8, f)))


def _edge_elementwise(g, edge_feats, we, be, half, eo_prev):
    bn = 10000
    nblk = E2 // bn
    off = half * nblk

    def body(g_ref, x_ref, w_ref, b_ref, *refs):
        eo_ref, t_ref = refs[-2], refs[-1]
        pe = (jnp.dot(x_ref[...], w_ref[...],
                      preferred_element_type=jnp.float32) + b_ref[0:1, :])
        eprod = g_ref[...] * pe
        eo_ref[...] = eprod
        row = lax.broadcasted_iota(jnp.int32, (HD, 16), 0)
        col = lax.broadcasted_iota(jnp.int32, (HD, 16), 1)
        bsel = (row // D == col).astype(jnp.float32)
        sums = jnp.dot(eprod, bsel, preferred_element_type=jnp.float32)
        t_ref[...] = jnp.exp(jnp.clip(sums, -CLIP, CLIP))

    in_specs = [
        pl.BlockSpec((bn, HD), lambda i: (i, 0)),
        pl.BlockSpec((bn, D_IN), lambda i: (i + off, 0)),
        pl.BlockSpec((D_IN, HD), lambda i: (0, 0)),
        pl.BlockSpec((8, HD), lambda i: (0, 0)),
    ]
    args = [g, edge_feats, we, jnp.broadcast_to(be, (8, HD))]
    aliases = {}
    if half == 1:
        in_specs.append(pl.BlockSpec(memory_space=pl.ANY))
        args.append(eo_prev)
        aliases = {4: 0}

    return pl.pallas_call(
        body,
        grid=(nblk,),
        in_specs=in_specs,
        out_specs=[
            pl.BlockSpec((bn, HD), lambda i: (i + off, 0)),
            pl.BlockSpec((bn, 16), lambda i: (i, 0)),
        ],
        out_shape=[
            jax.ShapeDtypeStruct((E, HD), jnp.float32),
            jax.ShapeDtypeStruct((E2, 16), jnp.float32),
        ],
        input_output_aliases=aliases,
    )(*args)


def _combine(p0, p1):
    bn = 2000

    def body(p0_ref, p1_ref, o_ref):
        p = p0_ref[0] + p0_ref[1] + p1_ref[0] + p1_ref[1]
        wv = p[:, :HD]
        z = p[:, HD:]
        col = lax.broadcasted_iota(jnp.int32, (16, HD), 1)
        row = lax.broadcasted_iota(jnp.int32, (16, HD), 0)
        sel = ((col // D == row) & (row < H)).astype(jnp.float32)
        zrep = jnp.dot(z, sel, preferred_element_type=jnp.float32)
        o_ref[...] = wv / (zrep + 1e-6)

    return pl.pallas_call(
        body,
        grid=(N // bn,),
        in_specs=[
            pl.BlockSpec((NC, bn, ACC_W), lambda i: (0, i, 0)),
            pl.BlockSpec((NC, bn, ACC_W), lambda i: (0, i, 0)),
        ],
        out_specs=pl.BlockSpec((bn, HD), lambda i: (i, 0)),
        out_shape=jax.ShapeDtypeStruct((N, HD), jnp.float32),
    )(p0, p1)


def _sc_mesh():
    return plsc.VectorSubcoreMesh(
        core_axis_name="c", subcore_axis_name="s",
        num_cores=NC, num_subcores=NS)


@functools.lru_cache(maxsize=2)
def _make_sc_score_kernel(half):
    @functools.partial(
        pl.kernel,
        out_type=jax.ShapeDtypeStruct((E2, HD), jnp.float32),
        mesh=_sc_mesh(),
        compiler_params=pltpu.CompilerParams(
            use_tc_tiling_on_sc=False, needs_layout_passes=False),
        scratch_types=[
            [pltpu.VMEM((CA,), jnp.int32) for _ in range(4)],
            [pltpu.VMEM((CA,), jnp.int32) for _ in range(4)],
            [pltpu.VMEM((CA, HD), jnp.float32) for _ in range(4)],
            [pltpu.VMEM((CA, HD), jnp.float32) for _ in range(4)],
            [pltpu.VMEM((CA, HD), jnp.float32) for _ in range(4)],
            [pltpu.SemaphoreType.DMA for _ in range(4)],
            [pltpu.SemaphoreType.DMA for _ in range(4)],
            [pltpu.SemaphoreType.DMA for _ in range(4)],
            [pltpu.SemaphoreType.DMA for _ in range(4)],
            [pltpu.SemaphoreType.DMA for _ in range(4)],
        ],
    )
    def sc_score_kernel(kh, qh, ei, g_hbm,
                        src_v, dst_v, k_v, q_v, g_v,
                        sem_is, sem_id, sem_k, sem_q, sem_g):
        cid = lax.axis_index("c")
        sid = lax.axis_index("s")
        wbase = (cid * NS + sid) * EPW
        ebase = half * E2 + wbase

        def issue_idx(n, p):
            eb = pl.multiple_of(ebase + n * CA, 8)
            pltpu.async_copy(ei.at[0, pl.ds(eb, CA)], src_v[p], sem_is[p])
            pltpu.async_copy(ei.at[1, pl.ds(eb, CA)], dst_v[p], sem_id[p])

        def drain_idx(p):
            pltpu.make_async_copy(
                ei.at[0, pl.ds(0, CA)], src_v[p], sem_is[p]).wait()
            pltpu.make_async_copy(
                ei.at[1, pl.ds(0, CA)], dst_v[p], sem_id[p]).wait()

        def issue_gathers(p):
            pltpu.async_copy(kh.at[src_v[p]], k_v[p], sem_k[p])
            pltpu.async_copy(qh.at[dst_v[p]], q_v[p], sem_q[p])

        def drain_gathers(p):
            pltpu.make_async_copy(kh.at[src_v[p]], k_v[p], sem_k[p]).wait()
            pltpu.make_async_copy(qh.at[dst_v[p]], q_v[p], sem_q[p]).wait()

        def drain_gout(p):
            pltpu.make_async_copy(
                g_v[p], g_hbm.at[pl.ds(0, CA), :], sem_g[p]).wait()

        def compute_chunk(p):
            kb, qb, gb = k_v[p], q_v[p], g_v[p]

            @plsc.parallel_loop(0, CA, unroll=4)
            def _(c):
                for h in range(H):
                    sl = pl.ds(h * D, D)
                    kq = kb[c, sl] * qb[c, sl]
                    gb[c, sl] = jnp.clip(kq * 0.25, -CLIP, CLIP)

        for n in range(4):
            issue_idx(n, n)
        for n in range(3):
            drain_idx(n)
            issue_gathers(n)

        def quad_body(gi, _):
            for b in range(4):
                j = 4 * gi + b

                @pl.when(j < CHUNKS_A)
                def _():
                    gob = pl.multiple_of(wbase + j * CA, 8)
                    drain_gathers(b)

                    @pl.when(j >= 4)
                    def _():
                        drain_gout(b)

                    @pl.when(j + 3 < CHUNKS_A)
                    def _():
                        drain_idx((b + 3) % 4)
                        issue_gathers((b + 3) % 4)

                    @pl.when(j + 4 < CHUNKS_A)
                    def _():
                        issue_idx(j + 4, b)

                    compute_chunk(b)
                    pltpu.async_copy(
                        g_v[b], g_hbm.at[pl.ds(gob, CA), :], sem_g[b])
            return 0

        lax.fori_loop(0, -(-CHUNKS_A // 4), quad_body, 0)
        for b in range(4):
            drain_gout(b)

    return sc_score_kernel


@functools.lru_cache(maxsize=2)
def _make_sc_scatter_kernel(half):
    @functools.partial(
        pl.kernel,
        out_type=jax.ShapeDtypeStruct((NC, N, ACC_W), jnp.float32),
        mesh=_sc_mesh(),
        compiler_params=pltpu.CompilerParams(
            use_tc_tiling_on_sc=False, needs_layout_passes=False),
        scratch_types=[
            [pltpu.VMEM((CB,), jnp.int32) for _ in range(2)],
            [pltpu.VMEM((CB,), jnp.int32) for _ in range(2)],
            [pltpu.VMEM((CB,), jnp.int32) for _ in range(2)],
            [pltpu.VMEM((CB, HD), jnp.float32) for _ in range(2)],
            [pltpu.VMEM((CB, 16), jnp.float32) for _ in range(2)],
            [pltpu.VMEM((CB, ACC_W), jnp.float32) for _ in range(2)],
            [pltpu.VMEM((RCHUNK, ACC_W), jnp.float32) for _ in range(2)],
            pltpu.VMEM_SHARED((N, ACC_W), jnp.float32),
            [pltpu.SemaphoreType.DMA for _ in range(2)],
            [pltpu.SemaphoreType.DMA for _ in range(2)],
            [pltpu.SemaphoreType.DMA for _ in range(2)],
            [pltpu.SemaphoreType.DMA for _ in range(2)],
            [pltpu.SemaphoreType.DMA for _ in range(2)],
            [pltpu.SemaphoreType.DMA for _ in range(2)],
        ],
    )
    def sc_scatter_kernel(vh, t, ei, wv_hbm,
                          src_v, dst_v, dst_sc, v_v, t_v, wv_v,
                          zb, acc, sem_io,
                          sem_is, sem_id, sem_v, sem_t, sem_sc):
        cid = lax.axis_index("c")
        sid = lax.axis_index("s")
        wbase = (cid * NS + sid) * EPW
        ebase = half * E2 + wbase

        zeros16 = jnp.zeros((16,), jnp.float32)

        def zero_row(r, _):
            for h in range(ACC_W // D):
                zb[0][r, pl.ds(h * D, D)] = zeros16
            return 0

        lax.fori_loop(0, RCHUNK, zero_row, 0)

        def init_body(j, _):
            row = j * NS + sid

            @pl.when(row < NRC)
            def _():
                base = pl.multiple_of(row * RCHUNK, 8)
                pltpu.async_copy(
                    zb[0], acc.at[pl.ds(base, RCHUNK), :], sem_io[0])
            return 0

        lax.fori_loop(0, NRC_PT, init_body, 0)

        def init_drain(j, _):
            row = j * NS + sid

            @pl.when(row < NRC)
            def _():
                pltpu.make_async_copy(
                    zb[0], acc.at[pl.ds(0, RCHUNK), :], sem_io[0]).wait()
            return 0

        lax.fori_loop(0, NRC_PT, init_drain, 0)
        plsc.subcore_barrier()

        def issue_idx(n, p):
            eb = pl.multiple_of(ebase + n * CB, 8)
            pltpu.async_copy(ei.at[0, pl.ds(eb, CB)], src_v[p], sem_is[p])
            pltpu.async_copy(ei.at[1, pl.ds(eb, CB)], dst_v[p], sem_id[p])

        def drain_idx(p):
            pltpu.make_async_copy(
                ei.at[0, pl.ds(0, CB)], src_v[p], sem_is[p]).wait()
            pltpu.make_async_copy(
                ei.at[1, pl.ds(0, CB)], dst_v[p], sem_id[p]).wait()

        def issue_gathers(n, p):
            tb = pl.multiple_of(wbase + n * CB, 8)
            pltpu.async_copy(vh.at[src_v[p]], v_v[p], sem_v[p])
            pltpu.async_copy(t.at[pl.ds(tb, CB), :], t_v[p], sem_t[p])

        def drain_gathers(p):
            pltpu.make_async_copy(vh.at[src_v[p]], v_v[p], sem_v[p]).wait()
            pltpu.make_async_copy(
                t.at[pl.ds(0, CB), :], t_v[p], sem_t[p]).wait()

        def drain_scat(p):
            pltpu.make_async_copy(
                wv_v[p], acc.at[dst_sc[p]], sem_sc[p]).wait()

        def compute_chunk(p):
            vb, tb, wvb = v_v[p], t_v[p], wv_v[p]

            @plsc.parallel_loop(0, CB, unroll=4)
            def _(c):
                trow = tb[c, :]
                for h in range(H):
                    th = trow.at[
                        jnp.full((16,), h, jnp.int32)].get(
                            mode="promise_in_bounds")
                    wvb[c, pl.ds(h * D, D)] = vb[c, pl.ds(h * D, D)] * th
                wvb[c, pl.ds(HD, 16)] = trow

        issue_idx(0, 0)
        issue_idx(1, 1)
        drain_idx(0)
        issue_gathers(0, 0)

        def pair_body(gi, _):
            for b in range(2):
                p = b
                q = 1 - b
                j = 2 * gi + b

                @pl.when(j < CHUNKS_B)
                def _():
                    drain_gathers(p)

                    @pl.when(j >= 2)
                    def _():
                        drain_scat(p)

                    for o in (0, 16, CB - 16):
                        dst_sc[p][pl.ds(o, 16)] = dst_v[p][pl.ds(o, 16)]

                    @pl.when(j + 1 < CHUNKS_B)
                    def _():
                        drain_idx(q)
                        issue_gathers(j + 1, q)

                    @pl.when(j + 2 < CHUNKS_B)
                    def _():
                        issue_idx(j + 2, p)

                    compute_chunk(p)
                    pltpu.async_copy(
                        wv_v[p], acc.at[dst_sc[p]], sem_sc[p], add=True)
            return 0

        lax.fori_loop(0, -(-CHUNKS_B // 2), pair_body, 0)
        drain_scat(0)
        drain_scat(1)
        plsc.subcore_barrier()

        def out_drain(j):
            pltpu.make_async_copy(
                zb[j % 2], wv_hbm.at[cid, pl.ds(0, RCHUNK), :],
                sem_io[j % 2]).wait()

        for j in range(NRC_PT):
            row = j * NS + sid

            @pl.when(row < NRC)
            def _():
                if j >= 2:
                    out_drain(j - 2)
                base = pl.multiple_of(row * RCHUNK, 8)
                pltpu.sync_copy(acc.at[pl.ds(base, RCHUNK), :], zb[j % 2])
                pltpu.async_copy(
                    zb[j % 2], wv_hbm.at[cid, pl.ds(base, RCHUNK), :],
                    sem_io[j % 2])

        for j in range(NRC_PT - 4, NRC_PT):
            row = j * NS + sid
            nxt = (j + 2) * NS + sid
            issued = row < NRC
            not_drained = (j + 2 >= NRC_PT) | (nxt >= NRC)

            @pl.when(issued & not_drained)
            def _():
                out_drain(j)

    return sc_scatter_kernel


def kernel(node_feats, edge_feats, edge_index, Wq, bq, Wk, bk, Wv, bv, We, be):
    w_qkv = jnp.concatenate([Wq, Wk, Wv], axis=1)
    b_qkv = jnp.concatenate([bq, bk, bv], axis=0)
    qkv = _matmul_bias(node_feats, w_qkv, b_qkv, 2000)
    q_h = qkv[:, :HD]
    k_h = qkv[:, HD:2 * HD]
    v_h = qkv[:, 2 * HD:]

    g0 = _make_sc_score_kernel(0)(k_h, q_h, edge_index)
    g1 = _make_sc_score_kernel(1)(k_h, q_h, edge_index)
    eo0, t0 = _edge_elementwise(g0, edge_feats, We, be, 0, None)
    e_out, t1 = _edge_elementwise(g1, edge_feats, We, be, 1, eo0)
    p0 = _make_sc_scatter_kernel(0)(v_h, t0, edge_index)
    p1 = _make_sc_scatter_kernel(1)(v_h, t1, edge_index)
    h_out = _combine(p0, p1)
    return (h_out.reshape(N, H, D), e_out.reshape(E, H, D))
